# baseline (speedup 1.0000x reference)
import functools

import jax
import jax.numpy as jnp
from jax import lax
from jax.experimental import pallas as pl
from jax.experimental.pallas import tpu as pltpu
from jax.experimental.pallas import tpu_sc as plsc

SMOOTH = 1e-05

N, C, H, W = 4, 21, 512, 512
NW = 32
RPW = H // 8
NCHUNK = 8
BAND = 8
NBIN = 32
HSIZE = 2 * NBIN * 16

_mesh = plsc.VectorSubcoreMesh(core_axis_name="c", subcore_axis_name="s")


@functools.partial(
    pl.kernel,
    out_type=jax.ShapeDtypeStruct((NW, HSIZE), jnp.int32),
    mesh=_mesh,
    compiler_params=pltpu.CompilerParams(needs_layout_passes=False),
    scratch_types=[
        pltpu.VMEM((C, BAND, W), jnp.float32),
        pltpu.VMEM((BAND, W), jnp.int32),
        pltpu.VMEM((HSIZE,), jnp.int32),
        pltpu.SemaphoreType.DMA,
    ],
)
def _sc_hist(x_hbm, t_hbm, out_hbm, xbuf, tbuf, hist, sem):
    wid = lax.axis_index("s") * 2 + lax.axis_index("c")
    n = wid // 8
    r0 = (wid % 8) * RPW

    zeros16 = jnp.zeros((16,), jnp.int32)
    ones16 = jnp.ones((16,), jnp.int32)
    lane = lax.iota(jnp.int32, 16)

    def _zero(k, carry):
        hist[pl.ds(k * 16, 16)] = zeros16
        return carry

    lax.fori_loop(0, HSIZE // 16, _zero, 0)

    for chunk in range(NCHUNK):
        h0 = r0 + chunk * BAND
        copies = [
            pltpu.async_copy(x_hbm.at[n, c, pl.ds(h0, BAND), :],
                             xbuf.at[c], sem)
            for c in range(C)
        ]
        tcopy = pltpu.async_copy(t_hbm.at[n, pl.ds(h0, BAND), :], tbuf, sem)
        for cp in copies:
            cp.wait()
        tcopy.wait()

        def _row(row, carry):
            def _vec(j, carry2):
                t16 = tbuf[row, pl.ds(j * 16, 16)]
                xs = [xbuf[c, row, pl.ds(j * 16, 16)] for c in range(C)]
                ms = xs
                while len(ms) > 1:
                    ms = [jnp.maximum(ms[k], ms[k + 1])
                          for k in range(0, len(ms) - 1, 2)] \
                        + ([ms[-1]] if len(ms) % 2 else [])
                m = ms[0]
                ps = [jnp.where(xs[c] == m, c, C) for c in range(C)]
                while len(ps) > 1:
                    ps = [jnp.minimum(ps[k], ps[k + 1])
                          for k in range(0, len(ps) - 1, 2)] \
                        + ([ps[-1]] if len(ps) % 2 else [])
                match = jnp.where(ps[0] == t16, ones16, zeros16)
                idx = t16 * 16 + lane
                plsc.addupdate_scatter(hist, [idx], ones16)
                plsc.addupdate_scatter(hist, [idx + NBIN * 16], match)
                return carry2

            return lax.fori_loop(0, W // 16, _vec, carry, unroll=2)

        lax.fori_loop(0, BAND, _row, 0)

    pltpu.sync_copy(hist, out_hbm.at[wid])


def _final_body(h_ref, out_ref):
    a = h_ref[...].astype(jnp.float32)
    tot = a[:, 0:NBIN * 16].reshape(NW, NBIN, 16)
    tp = a[:, NBIN * 16:].reshape(NW, NBIN, 16)
    tots = jnp.sum(tot, axis=2).reshape(N, 8, NBIN).sum(axis=1)
    tps = jnp.sum(tp, axis=2).reshape(N, 8, NBIN).sum(axis=1)
    rec = (tps + SMOOTH) / (tots + SMOOTH)
    cmask = lax.broadcasted_iota(jnp.int32, (N, NBIN), 1) < C
    s = jnp.sum(jnp.where(cmask, rec, 0.0))
    out_ref[0, 0] = 1.0 - s / (N * C)


def kernel(input, target):
    t = target.astype(jnp.int32)
    part = _sc_hist(input, t)
    out = pl.pallas_call(
        _final_body,
        out_specs=pl.BlockSpec(memory_space=pltpu.SMEM),
        out_shape=jax.ShapeDtypeStruct((1, 1), jnp.float32),
    )(part)
    return out[0, 0]


# speedup vs baseline: 2.3009x; 2.3009x over previous
import functools

import jax
import jax.numpy as jnp
from jax import lax
from jax.experimental import pallas as pl
from jax.experimental.pallas import tpu as pltpu
from jax.experimental.pallas import tpu_sc as plsc

SMOOTH = 1e-05

N, C, H, W = 4, 21, 512, 512
HS = 384
BH = 128
NB = HS // BH
CPAD = 24
SHIFT = 4096

NW = 32
RPW = (H - HS) // 8
BAND = 8
NCHUNK = RPW // BAND
NBIN = 32
HSIZE = 2 * NBIN * 16


def _tc_body(x_ref, t_ref, acc_ref):
    i = pl.program_id(0)

    @pl.when(i % NB == 0)
    def _init():
        acc_ref[...] = jnp.zeros((1, CPAD, 8, 128), jnp.int32)

    t = t_ref[0]
    m = x_ref[0, 0]
    pred = jnp.zeros((BH, W), jnp.int32)
    for c in range(1, C):
        xc = x_ref[0, c]
        gt = xc > m
        pred = jnp.where(gt, c, pred)
        m = jnp.maximum(xc, m)
    enc = jnp.where(pred == t, 1 + SHIFT, 1)
    for c in range(C):
        ec = jnp.where(t == c, enc, 0)
        p = ec[0:8, :]
        for s in range(1, BH // 8):
            p = p + ec[s * 8:(s + 1) * 8, :]
        q = ((p[:, 0:128] + p[:, 128:256])
             + (p[:, 256:384] + p[:, 384:512]))
        acc_ref[0, c] += q


def _tc_hist(x, t):
    return pl.pallas_call(
        _tc_body,
        grid=(N * NB,),
        in_specs=[
            pl.BlockSpec((1, C, BH, W), lambda i: (i // NB, 0, i % NB, 0)),
            pl.BlockSpec((1, BH, W), lambda i: (i // NB, i % NB, 0)),
        ],
        out_specs=pl.BlockSpec((1, CPAD, 8, 128), lambda i: (i // NB, 0, 0, 0)),
        out_shape=jax.ShapeDtypeStruct((N, CPAD, 8, 128), jnp.int32),
    )(x, t)


_mesh = plsc.VectorSubcoreMesh(core_axis_name="c", subcore_axis_name="s")


@functools.partial(
    pl.kernel,
    out_type=jax.ShapeDtypeStruct((NW, HSIZE), jnp.int32),
    mesh=_mesh,
    compiler_params=pltpu.CompilerParams(needs_layout_passes=False),
    scratch_types=[
        pltpu.VMEM((C, BAND, W), jnp.float32),
        pltpu.VMEM((BAND, W), jnp.int32),
        pltpu.VMEM((HSIZE,), jnp.int32),
        pltpu.SemaphoreType.DMA,
    ],
)
def _sc_hist(x_hbm, t_hbm, out_hbm, xbuf, tbuf, hist, sem):
    wid = lax.axis_index("s") * 2 + lax.axis_index("c")
    n = wid // 8
    r0 = HS + (wid % 8) * RPW

    zeros16 = jnp.zeros((16,), jnp.int32)
    ones16 = jnp.ones((16,), jnp.int32)
    lane = lax.iota(jnp.int32, 16)

    def _zero(k, carry):
        hist[pl.ds(k * 16, 16)] = zeros16
        return carry

    lax.fori_loop(0, HSIZE // 16, _zero, 0)

    for chunk in range(NCHUNK):
        h0 = r0 + chunk * BAND
        copies = [
            pltpu.async_copy(x_hbm.at[n, c, pl.ds(h0, BAND), :],
                             xbuf.at[c], sem)
            for c in range(C)
        ]
        tcopy = pltpu.async_copy(t_hbm.at[n, pl.ds(h0, BAND), :], tbuf, sem)
        for cp in copies:
            cp.wait()
        tcopy.wait()

        def _row(row, carry):
            def _vec(j, carry2):
                t16 = tbuf[row, pl.ds(j * 16, 16)]
                m = xbuf[0, row, pl.ds(j * 16, 16)]
                pred = zeros16
                for c in range(1, C):
                    xc = xbuf[c, row, pl.ds(j * 16, 16)]
                    gt = xc > m
                    pred = jnp.where(gt, c, pred)
                    m = jnp.where(gt, xc, m)
                match = jnp.where(pred == t16, ones16, zeros16)
                idx = t16 * 16 + lane
                plsc.addupdate_scatter(hist, [idx], ones16)
                plsc.addupdate_scatter(hist, [idx + NBIN * 16], match)
                return carry2

            return lax.fori_loop(0, W // 16, _vec, carry)

        lax.fori_loop(0, BAND, _row, 0)

    pltpu.sync_copy(hist, out_hbm.at[wid])


def _final_body(acc_ref, h_ref, out_ref):
    a = acc_ref[...]
    tp_tc = a // SHIFT
    tot_tc = a - tp_tc * SHIFT
    tps = jnp.sum(tp_tc.astype(jnp.float32), axis=(2, 3))
    tots = jnp.sum(tot_tc.astype(jnp.float32), axis=(2, 3))

    hsc = h_ref[...].astype(jnp.float32)
    tot_sc = hsc[:, 0:NBIN * 16].reshape(NW, NBIN, 16)
    tp_sc = hsc[:, NBIN * 16:].reshape(NW, NBIN, 16)
    tots_sc = jnp.sum(tot_sc, axis=2).reshape(N, 8, NBIN).sum(axis=1)
    tps_sc = jnp.sum(tp_sc, axis=2).reshape(N, 8, NBIN).sum(axis=1)

    tots = tots + tots_sc[:, 0:CPAD]
    tps = tps + tps_sc[:, 0:CPAD]
    rec = (tps + SMOOTH) / (tots + SMOOTH)
    cmask = lax.broadcasted_iota(jnp.int32, (N, CPAD), 1) < C
    s = jnp.sum(jnp.where(cmask, rec, 0.0))
    out_ref[0, 0] = 1.0 - s / (N * C)


def kernel(input, target):
    t = target.astype(jnp.int32)
    part_sc = _sc_hist(input, t)
    part_tc = _tc_hist(input, t)
    out = pl.pallas_call(
        _final_body,
        out_specs=pl.BlockSpec(memory_space=pltpu.SMEM),
        out_shape=jax.ShapeDtypeStruct((1, 1), jnp.float32),
    )(part_tc, part_sc)
    return out[0, 0]


# speedup vs baseline: 2.3252x; 1.0105x over previous
import functools

import jax
import jax.numpy as jnp
from jax import lax
from jax.experimental import pallas as pl
from jax.experimental.pallas import tpu as pltpu
from jax.experimental.pallas import tpu_sc as plsc

SMOOTH = 1e-05

N, C, H, W = 4, 21, 512, 512
HS = 448
BH = 224
NB = HS // BH
CPAD = 24
SHIFT = 4096

NW = 32
RPW = (H - HS) // 8
BAND = 8
NCHUNK = RPW // BAND
NBIN = 32
HSIZE = 2 * NBIN * 16


def _tc_body(x_ref, t_ref, acc_ref):
    i = pl.program_id(0)

    @pl.when(i % NB == 0)
    def _init():
        acc_ref[...] = jnp.zeros((1, CPAD, 8, 128), jnp.int32)

    t = t_ref[0]
    m = x_ref[0, 0]
    pred = jnp.zeros((BH, W), jnp.int32)
    for c in range(1, C):
        xc = x_ref[0, c]
        gt = xc > m
        pred = jnp.where(gt, c, pred)
        m = jnp.maximum(xc, m)
    enc = jnp.where(pred == t, 1 + SHIFT, 1)
    for c in range(C):
        ec = jnp.where(t == c, enc, 0)
        p = ec[0:8, :]
        for s in range(1, BH // 8):
            p = p + ec[s * 8:(s + 1) * 8, :]
        q = ((p[:, 0:128] + p[:, 128:256])
             + (p[:, 256:384] + p[:, 384:512]))
        acc_ref[0, c] += q


def _tc_hist(x, t):
    return pl.pallas_call(
        _tc_body,
        grid=(N * NB,),
        in_specs=[
            pl.BlockSpec((1, C, BH, W), lambda i: (i // NB, 0, i % NB, 0)),
            pl.BlockSpec((1, BH, W), lambda i: (i // NB, i % NB, 0)),
        ],
        out_specs=pl.BlockSpec((1, CPAD, 8, 128), lambda i: (i // NB, 0, 0, 0)),
        out_shape=jax.ShapeDtypeStruct((N, CPAD, 8, 128), jnp.int32),
    )(x, t)


_mesh = plsc.VectorSubcoreMesh(core_axis_name="c", subcore_axis_name="s")


@functools.partial(
    pl.kernel,
    out_type=jax.ShapeDtypeStruct((NW, HSIZE), jnp.int32),
    mesh=_mesh,
    compiler_params=pltpu.CompilerParams(needs_layout_passes=False),
    scratch_types=[
        pltpu.VMEM((C, BAND, W), jnp.float32),
        pltpu.VMEM((BAND, W), jnp.int32),
        pltpu.VMEM((HSIZE,), jnp.int32),
        pltpu.SemaphoreType.DMA,
    ],
)
def _sc_hist(x_hbm, t_hbm, out_hbm, xbuf, tbuf, hist, sem):
    wid = lax.axis_index("s") * 2 + lax.axis_index("c")
    n = wid // 8
    r0 = HS + (wid % 8) * RPW

    zeros16 = jnp.zeros((16,), jnp.int32)
    ones16 = jnp.ones((16,), jnp.int32)
    lane = lax.iota(jnp.int32, 16)

    def _zero(k, carry):
        hist[pl.ds(k * 16, 16)] = zeros16
        return carry

    lax.fori_loop(0, HSIZE // 16, _zero, 0)

    for chunk in range(NCHUNK):
        h0 = r0 + chunk * BAND
        copies = [
            pltpu.async_copy(x_hbm.at[n, c, pl.ds(h0, BAND), :],
                             xbuf.at[c], sem)
            for c in range(C)
        ]
        tcopy = pltpu.async_copy(t_hbm.at[n, pl.ds(h0, BAND), :], tbuf, sem)
        for cp in copies:
            cp.wait()
        tcopy.wait()

        def _row(row, carry):
            def _vec(j, carry2):
                t16 = tbuf[row, pl.ds(j * 16, 16)]
                m = xbuf[0, row, pl.ds(j * 16, 16)]
                pred = zeros16
                for c in range(1, C):
                    xc = xbuf[c, row, pl.ds(j * 16, 16)]
                    gt = xc > m
                    pred = jnp.where(gt, c, pred)
                    m = jnp.where(gt, xc, m)
                match = jnp.where(pred == t16, ones16, zeros16)
                idx = t16 * 16 + lane
                plsc.addupdate_scatter(hist, [idx], ones16)
                plsc.addupdate_scatter(hist, [idx + NBIN * 16], match)
                return carry2

            return lax.fori_loop(0, W // 16, _vec, carry)

        lax.fori_loop(0, BAND, _row, 0)

    pltpu.sync_copy(hist, out_hbm.at[wid])


def _final_body(acc_ref, h_ref, out_ref):
    a = acc_ref[...]
    tp_tc = a // SHIFT
    tot_tc = a - tp_tc * SHIFT
    tps = jnp.sum(tp_tc.astype(jnp.float32), axis=(2, 3))
    tots = jnp.sum(tot_tc.astype(jnp.float32), axis=(2, 3))

    hsc = h_ref[...].astype(jnp.float32)
    tot_sc = hsc[:, 0:NBIN * 16].reshape(NW, NBIN, 16)
    tp_sc = hsc[:, NBIN * 16:].reshape(NW, NBIN, 16)
    tots_sc = jnp.sum(tot_sc, axis=2).reshape(N, 8, NBIN).sum(axis=1)
    tps_sc = jnp.sum(tp_sc, axis=2).reshape(N, 8, NBIN).sum(axis=1)

    tots = tots + tots_sc[:, 0:CPAD]
    tps = tps + tps_sc[:, 0:CPAD]
    rec = (tps + SMOOTH) / (tots + SMOOTH)
    cmask = lax.broadcasted_iota(jnp.int32, (N, CPAD), 1) < C
    s = jnp.sum(jnp.where(cmask, rec, 0.0))
    out_ref[0, 0] = 1.0 - s / (N * C)


def kernel(input, target):
    t = target.astype(jnp.int32)
    part_sc = _sc_hist(input, t)
    part_tc = _tc_hist(input, t)
    out = pl.pallas_call(
        _final_body,
        out_specs=pl.BlockSpec(memory_space=pltpu.SMEM),
        out_shape=jax.ShapeDtypeStruct((1, 1), jnp.float32),
    )(part_tc, part_sc)
    return out[0, 0]
